# baseline (speedup 1.0000x reference)
import functools

import jax
import jax.numpy as jnp
from jax import lax
from jax.experimental import pallas as pl
from jax.experimental.pallas import tpu as pltpu
from jax.experimental.pallas import tpu_sc as plsc

VOCAB = 100000
DIM = 1024
BATCH = 4
SEQLEN = 2048
N = BATCH * SEQLEN

NC = 2
NS = 16
LANES = 16
NW = NC * NS
BPW = N // NW
CHUNK = 32
NCHUNK = BPW // CHUNK
SLICES = DIM // LANES


def _body(ids_hbm, word_hbm, pos_hbm, out_hbm, idx_v, word_v, pos_v, sem):
    wid = lax.axis_index("s") * NC + lax.axis_index("c")
    base = wid * BPW
    s0 = lax.rem(base, SEQLEN)

    pltpu.sync_copy(ids_hbm.at[pl.ds(base, BPW)], idx_v)

    def chunk(j, carry):
        off = j * CHUNK
        gather = pltpu.async_copy(
            word_hbm.at[idx_v.at[pl.ds(off, CHUNK)]], word_v, sem)
        pltpu.sync_copy(pos_hbm.at[pl.ds(s0 + off, CHUNK)], pos_v)
        gather.wait()

        def add_row(r, carry2):
            for c in range(SLICES):
                sl = pl.ds(c * LANES, LANES)
                word_v[r, sl] = word_v[r, sl] + pos_v[r, sl]
            return carry2

        lax.fori_loop(0, CHUNK, add_row, 0, unroll=False)
        pltpu.sync_copy(word_v, out_hbm.at[pl.ds(base + off, CHUNK)])
        return carry

    lax.fori_loop(0, NCHUNK, chunk, 0, unroll=False)


@jax.jit
def _embed(flat_ids, word_embeddings, position_embeddings):
    mesh = plsc.VectorSubcoreMesh(core_axis_name="c", subcore_axis_name="s")
    return pl.kernel(
        _body,
        out_type=jax.ShapeDtypeStruct((N, DIM), jnp.float32),
        mesh=mesh,
        scratch_types=[
            pltpu.VMEM((BPW,), jnp.int32),
            pltpu.VMEM((CHUNK, DIM), jnp.float32),
            pltpu.VMEM((CHUNK, DIM), jnp.float32),
            pltpu.SemaphoreType.DMA,
        ],
    )(flat_ids, word_embeddings, position_embeddings)


def kernel(input_ids, word_embeddings, position_embeddings):
    flat_ids = input_ids.reshape(-1).astype(jnp.int32)
    out = _embed(flat_ids, word_embeddings, position_embeddings)
    return out.reshape(BATCH, SEQLEN, DIM)


# speedup vs baseline: 1.1658x; 1.1658x over previous
import jax
import jax.numpy as jnp
from jax import lax
from jax.experimental import pallas as pl
from jax.experimental.pallas import tpu as pltpu
from jax.experimental.pallas import tpu_sc as plsc

DIM = 1024
BATCH = 4
SEQLEN = 2048
N = BATCH * SEQLEN

NC = 2
NS = 16
LANES = 16
NW = NC * NS
SPW = SEQLEN // NW
CHUNK = 16
JCHUNK = SPW // CHUNK
NCHUNKS = BATCH * JCHUNK
SLICES = DIM // LANES


def _body(ids_hbm, word_hbm, pos_hbm, out_hbm,
          idx_v, pos_v, buf0, buf1, g0, g1, st0, st1):
    wid = lax.axis_index("s") * NC + lax.axis_index("c")
    s0 = wid * SPW

    pltpu.sync_copy(pos_hbm.at[pl.ds(s0, SPW)], pos_v)
    for b in range(BATCH):
        pltpu.sync_copy(ids_hbm.at[pl.ds(b * SEQLEN + s0, SPW)],
                        idx_v.at[pl.ds(b * SPW, SPW)])

    bufs = (buf0, buf1)
    gsems = (g0, g1)
    ssems = (st0, st1)

    def start_gather(k):
        b, j = divmod(k, JCHUNK)
        return pltpu.async_copy(
            word_hbm.at[idx_v.at[pl.ds(b * SPW + j * CHUNK, CHUNK)]],
            bufs[k % 2], gsems[k % 2])

    def start_store(k):
        b, j = divmod(k, JCHUNK)
        return pltpu.async_copy(
            bufs[k % 2],
            out_hbm.at[pl.ds(b * SEQLEN + s0 + j * CHUNK, CHUNK)],
            ssems[k % 2])

    gathers = {0: start_gather(0)}
    stores = {}
    for k in range(NCHUNKS):
        cur = k % 2
        if k + 1 < NCHUNKS:
            if k - 1 in stores:
                stores[k - 1].wait()
            gathers[k + 1] = start_gather(k + 1)
        gathers[k].wait()

        _, j = divmod(k, JCHUNK)
        buf = bufs[cur]

        def add_row(r, carry, buf=buf, j=j):
            for c in range(SLICES):
                sl = pl.ds(c * LANES, LANES)
                plsc.addupdate(buf.at[r, sl], pos_v[j * CHUNK + r, sl])
            return carry

        lax.fori_loop(0, CHUNK, add_row, 0, unroll=False)
        stores[k] = start_store(k)
    stores[NCHUNKS - 2].wait()
    stores[NCHUNKS - 1].wait()


@jax.jit
def _embed(flat_ids, word_embeddings, position_embeddings):
    mesh = plsc.VectorSubcoreMesh(core_axis_name="c", subcore_axis_name="s")
    return pl.kernel(
        _body,
        out_type=jax.ShapeDtypeStruct((N, DIM), jnp.float32),
        mesh=mesh,
        scratch_types=[
            pltpu.VMEM((BATCH * SPW,), jnp.int32),
            pltpu.VMEM((SPW, DIM), jnp.float32),
            pltpu.VMEM((CHUNK, DIM), jnp.float32),
            pltpu.VMEM((CHUNK, DIM), jnp.float32),
            pltpu.SemaphoreType.DMA,
            pltpu.SemaphoreType.DMA,
            pltpu.SemaphoreType.DMA,
            pltpu.SemaphoreType.DMA,
        ],
    )(flat_ids, word_embeddings, position_embeddings)


def kernel(input_ids, word_embeddings, position_embeddings):
    flat_ids = input_ids.reshape(-1).astype(jnp.int32)
    out = _embed(flat_ids, word_embeddings, position_embeddings)
    return out.reshape(BATCH, SEQLEN, DIM)


# speedup vs baseline: 1.4591x; 1.2516x over previous
import jax
import jax.numpy as jnp
from jax import lax
from jax.experimental import pallas as pl
from jax.experimental.pallas import tpu as pltpu
from jax.experimental.pallas import tpu_sc as plsc

DIM = 1024
BATCH = 4
SEQLEN = 2048
N = BATCH * SEQLEN

NC = 2
NS = 16
LANES = 16
NW = NC * NS
SPW = SEQLEN // NW
CHUNK = 8
NQ = SPW // CHUNK
PQ = 16
SLICES = DIM // LANES


def _body(ids_hbm, word_hbm, pos_hbm, out_hbm,
          idx_v, pos0, pos1, b0a, b1a, b2a, b3a, b0b, b1b, b2b, b3b,
          ga, gb, sa, sb, psem):
    wid = lax.axis_index("s") * NC + lax.axis_index("c")
    s0 = wid * SPW

    for b in range(BATCH):
        pltpu.sync_copy(ids_hbm.at[pl.ds(b * SEQLEN + s0, SPW)],
                        idx_v.at[pl.ds(b * SPW, SPW)])

    bufsets = ((b0a, b1a, b2a, b3a), (b0b, b1b, b2b, b3b))
    gsems = (ga, gb)
    ssems = (sa, sb)
    posbufs = (pos0, pos1)

    def start_gathers(q):
        p = q % 2
        return [pltpu.async_copy(
            word_hbm.at[idx_v.at[pl.ds(b * SPW + q * CHUNK, CHUNK)]],
            bufsets[p][b], gsems[p]) for b in range(BATCH)]

    def start_stores(q):
        p = q % 2
        return [pltpu.async_copy(
            bufsets[p][b],
            out_hbm.at[pl.ds(b * SEQLEN + s0 + q * CHUNK, CHUNK)],
            ssems[p]) for b in range(BATCH)]

    def start_posload(t):
        return pltpu.async_copy(
            pos_hbm.at[pl.ds(s0 + t * PQ, PQ)], posbufs[t % 2], psem)

    posloads = {0: start_posload(0)}
    gathers = {0: start_gathers(0)}
    stores = {}
    for q in range(NQ):
        p = q % 2
        if q + 1 < NQ:
            if q - 1 in stores:
                for st in stores[q - 1]:
                    st.wait()
            gathers[q + 1] = start_gathers(q + 1)
        if q % 2 == 0 and q // 2 + 1 < NQ // 2:
            posloads[q // 2 + 1] = start_posload(q // 2 + 1)
        for g in gathers[q]:
            g.wait()
        if q % 2 == 0:
            posloads[q // 2].wait()

        pbuf = posbufs[(q // 2) % 2]
        prow0 = (q % 2) * CHUNK
        bufs = bufsets[p]

        def add_row(r, carry, bufs=bufs, pbuf=pbuf, prow0=prow0):
            for c in range(SLICES):
                sl = pl.ds(c * LANES, LANES)
                pv = pbuf[prow0 + r, sl]
                for b in range(BATCH):
                    plsc.addupdate(bufs[b].at[r, sl], pv)
            return carry

        lax.fori_loop(0, CHUNK, add_row, 0, unroll=False)
        stores[q] = start_stores(q)
    for q in (NQ - 2, NQ - 1):
        for st in stores[q]:
            st.wait()


@jax.jit
def _embed(flat_ids, word_embeddings, position_embeddings):
    mesh = plsc.VectorSubcoreMesh(core_axis_name="c", subcore_axis_name="s")
    buf = pltpu.VMEM((CHUNK, DIM), jnp.float32)
    return pl.kernel(
        _body,
        out_type=jax.ShapeDtypeStruct((N, DIM), jnp.float32),
        mesh=mesh,
        scratch_types=[
            pltpu.VMEM((BATCH * SPW,), jnp.int32),
            pltpu.VMEM((PQ, DIM), jnp.float32),
            pltpu.VMEM((PQ, DIM), jnp.float32),
            buf, buf, buf, buf, buf, buf, buf, buf,
            pltpu.SemaphoreType.DMA,
            pltpu.SemaphoreType.DMA,
            pltpu.SemaphoreType.DMA,
            pltpu.SemaphoreType.DMA,
            pltpu.SemaphoreType.DMA,
        ],
    )(flat_ids, word_embeddings, position_embeddings)


def kernel(input_ids, word_embeddings, position_embeddings):
    flat_ids = input_ids.reshape(-1).astype(jnp.int32)
    out = _embed(flat_ids, word_embeddings, position_embeddings)
    return out.reshape(BATCH, SEQLEN, DIM)


# speedup vs baseline: 1.4626x; 1.0024x over previous
import jax
import jax.numpy as jnp
from jax import lax
from jax.experimental import pallas as pl
from jax.experimental.pallas import tpu as pltpu
from jax.experimental.pallas import tpu_sc as plsc

DIM = 1024
BATCH = 4
SEQLEN = 2048
N = BATCH * SEQLEN

NC = 2
NS = 16
LANES = 16
NW = NC * NS
SPW = SEQLEN // NW
CHUNK = 8
NQ = SPW // CHUNK
NSET = 3
SLICES = DIM // LANES


def _body(ids_hbm, word_hbm, pos_hbm, out_hbm, idx_v,
          p0, p1, p2,
          b00, b01, b02, b03, b10, b11, b12, b13, b20, b21, b22, b23,
          g0, g1, g2, s0, s1, s2):
    wid = lax.axis_index("s") * NC + lax.axis_index("c")
    seq0 = wid * SPW

    for b in range(BATCH):
        pltpu.sync_copy(ids_hbm.at[pl.ds(b * SEQLEN + seq0, SPW)],
                        idx_v.at[pl.ds(b * SPW, SPW)])

    bufsets = ((b00, b01, b02, b03), (b10, b11, b12, b13),
               (b20, b21, b22, b23))
    posbufs = (p0, p1, p2)
    gsems = (g0, g1, g2)
    ssems = (s0, s1, s2)

    def start_set(q):
        p = q % NSET
        cps = [pltpu.async_copy(
            word_hbm.at[idx_v.at[pl.ds(b * SPW + q * CHUNK, CHUNK)]],
            bufsets[p][b], gsems[p]) for b in range(BATCH)]
        cps.append(pltpu.async_copy(
            pos_hbm.at[pl.ds(seq0 + q * CHUNK, CHUNK)], posbufs[p], gsems[p]))
        return cps

    def start_stores(q):
        p = q % NSET
        return [pltpu.async_copy(
            bufsets[p][b],
            out_hbm.at[pl.ds(b * SEQLEN + seq0 + q * CHUNK, CHUNK)],
            ssems[p]) for b in range(BATCH)]

    inflight = {0: start_set(0), 1: start_set(1)}
    stores = {}
    for q in range(NQ):
        if q + 2 < NQ:
            if q - 1 in stores:
                for st in stores[q - 1]:
                    st.wait()
            inflight[q + 2] = start_set(q + 2)
        for c in inflight[q]:
            c.wait()

        pbuf = posbufs[q % NSET]
        bufs = bufsets[q % NSET]

        def add_row(r, carry, bufs=bufs, pbuf=pbuf):
            for c in range(SLICES):
                sl = pl.ds(c * LANES, LANES)
                pv = pbuf[r, sl]
                for b in range(BATCH):
                    plsc.addupdate(bufs[b].at[r, sl], pv)
            return carry

        lax.fori_loop(0, CHUNK, add_row, 0, unroll=False)
        stores[q] = start_stores(q)
    for q in (NQ - 3, NQ - 2, NQ - 1):
        for st in stores[q]:
            st.wait()


@jax.jit
def _embed(flat_ids, word_embeddings, position_embeddings):
    mesh = plsc.VectorSubcoreMesh(core_axis_name="c", subcore_axis_name="s")
    buf = pltpu.VMEM((CHUNK, DIM), jnp.float32)
    return pl.kernel(
        _body,
        out_type=jax.ShapeDtypeStruct((N, DIM), jnp.float32),
        mesh=mesh,
        scratch_types=[
            pltpu.VMEM((BATCH * SPW,), jnp.int32),
            buf, buf, buf,
            buf, buf, buf, buf, buf, buf, buf, buf, buf, buf, buf, buf,
            pltpu.SemaphoreType.DMA, pltpu.SemaphoreType.DMA,
            pltpu.SemaphoreType.DMA, pltpu.SemaphoreType.DMA,
            pltpu.SemaphoreType.DMA, pltpu.SemaphoreType.DMA,
        ],
    )(flat_ids, word_embeddings, position_embeddings)


def kernel(input_ids, word_embeddings, position_embeddings):
    flat_ids = input_ids.reshape(-1).astype(jnp.int32)
    out = _embed(flat_ids, word_embeddings, position_embeddings)
    return out.reshape(BATCH, SEQLEN, DIM)


# speedup vs baseline: 1.4636x; 1.0007x over previous
import jax
import jax.numpy as jnp
from jax import lax
from jax.experimental import pallas as pl
from jax.experimental.pallas import tpu as pltpu
from jax.experimental.pallas import tpu_sc as plsc

DIM = 1024
BATCH = 4
SEQLEN = 2048
N = BATCH * SEQLEN

NC = 2
NS = 16
LANES = 16
NW = NC * NS
SPW = SEQLEN // NW
CHUNK = 8
NQ = SPW // CHUNK
NSET = 3
SLICES = DIM // LANES


def _body(ids_hbm, word_hbm, pos_hbm, out_hbm, idx_v,
          p0, p1, p2,
          b00, b01, b02, b03, b10, b11, b12, b13, b20, b21, b22, b23,
          g0, g1, g2, s0, s1, s2):
    wid = lax.axis_index("s") * NC + lax.axis_index("c")
    seq0 = wid * SPW

    for b in range(BATCH):
        pltpu.sync_copy(ids_hbm.at[pl.ds(b * SEQLEN + seq0, SPW)],
                        idx_v.at[pl.ds(b * SPW, SPW)])

    bufsets = ((b00, b01, b02, b03), (b10, b11, b12, b13),
               (b20, b21, b22, b23))
    posbufs = (p0, p1, p2)
    gsems = (g0, g1, g2)
    ssems = (s0, s1, s2)

    def start_set(q):
        p = q % NSET
        cps = [pltpu.async_copy(
            word_hbm.at[idx_v.at[pl.ds(b * SPW + q * CHUNK, CHUNK)]],
            bufsets[p][b], gsems[p]) for b in range(BATCH)]
        cps.append(pltpu.async_copy(
            pos_hbm.at[pl.ds(seq0 + q * CHUNK, CHUNK)], posbufs[p], gsems[p]))
        return cps

    def start_stores(q):
        p = q % NSET
        return [pltpu.async_copy(
            bufsets[p][b],
            out_hbm.at[pl.ds(b * SEQLEN + seq0 + q * CHUNK, CHUNK)],
            ssems[p]) for b in range(BATCH)]

    inflight = {0: start_set(0), 1: start_set(1)}
    stores = {}
    for q in range(NQ):
        for c in inflight[q]:
            c.wait()

        pbuf = posbufs[q % NSET]
        bufs = bufsets[q % NSET]

        def add_row(r, carry, bufs=bufs, pbuf=pbuf):
            for c in range(SLICES):
                sl = pl.ds(c * LANES, LANES)
                pv = pbuf[r, sl]
                for b in range(BATCH):
                    plsc.addupdate(bufs[b].at[r, sl], pv)
            return carry

        lax.fori_loop(0, CHUNK, add_row, 0, unroll=False)
        stores[q] = start_stores(q)
        if q + 2 < NQ:
            if q - 1 in stores:
                for st in stores[q - 1]:
                    st.wait()
            inflight[q + 2] = start_set(q + 2)
    for q in (NQ - 3, NQ - 2, NQ - 1):
        for st in stores[q]:
            st.wait()


@jax.jit
def _embed(flat_ids, word_embeddings, position_embeddings):
    mesh = plsc.VectorSubcoreMesh(core_axis_name="c", subcore_axis_name="s")
    buf = pltpu.VMEM((CHUNK, DIM), jnp.float32)
    return pl.kernel(
        _body,
        out_type=jax.ShapeDtypeStruct((N, DIM), jnp.float32),
        mesh=mesh,
        scratch_types=[
            pltpu.VMEM((BATCH * SPW,), jnp.int32),
            buf, buf, buf,
            buf, buf, buf, buf, buf, buf, buf, buf, buf, buf, buf, buf,
            pltpu.SemaphoreType.DMA, pltpu.SemaphoreType.DMA,
            pltpu.SemaphoreType.DMA, pltpu.SemaphoreType.DMA,
            pltpu.SemaphoreType.DMA, pltpu.SemaphoreType.DMA,
        ],
    )(flat_ids, word_embeddings, position_embeddings)


def kernel(input_ids, word_embeddings, position_embeddings):
    flat_ids = input_ids.reshape(-1).astype(jnp.int32)
    out = _embed(flat_ids, word_embeddings, position_embeddings)
    return out.reshape(BATCH, SEQLEN, DIM)


# speedup vs baseline: 1.6601x; 1.1343x over previous
import jax
import jax.numpy as jnp
from jax import lax
from jax.experimental import pallas as pl
from jax.experimental.pallas import tpu as pltpu
from jax.experimental.pallas import tpu_sc as plsc

DIM = 1024
BATCH = 4
SEQLEN = 2048
N = BATCH * SEQLEN

NC = 2
NS = 16
LANES = 16
NW = NC * NS
SPW = SEQLEN // NW
CHUNK = 8
NQ = SPW // CHUNK
SLICES = DIM // LANES


def _body(ids_hbm, word_hbm, pos_hbm, out_hbm, idx_v,
          p0, p1, b00, b01, b02, b03, b10, b11, b12, b13,
          g0, g1, s0, s1):
    wid = lax.axis_index("s") * NC + lax.axis_index("c")
    seq0 = wid * SPW

    for b in range(BATCH):
        pltpu.sync_copy(ids_hbm.at[pl.ds(b * SEQLEN + seq0, SPW)],
                        idx_v.at[pl.ds(b * SPW, SPW)])

    bufsets = ((b00, b01, b02, b03), (b10, b11, b12, b13))
    posbufs = (p0, p1)
    gsems = (g0, g1)
    ssems = (s0, s1)

    def start_set(q, p):
        for b in range(BATCH):
            pltpu.async_copy(
                word_hbm.at[idx_v.at[pl.ds(b * SPW + q * CHUNK, CHUNK)]],
                bufsets[p][b], gsems[p])
        pltpu.async_copy(
            pos_hbm.at[pl.ds(seq0 + q * CHUNK, CHUNK)], posbufs[p], gsems[p])

    def drain_gathers(p):
        for b in range(BATCH):
            pltpu.make_async_copy(
                out_hbm.at[pl.ds(0, CHUNK)], bufsets[p][b], gsems[p]).wait()
        pltpu.make_async_copy(
            out_hbm.at[pl.ds(0, CHUNK)], posbufs[p], gsems[p]).wait()

    def start_stores(q, p):
        for b in range(BATCH):
            pltpu.async_copy(
                bufsets[p][b],
                out_hbm.at[pl.ds(b * SEQLEN + seq0 + q * CHUNK, CHUNK)],
                ssems[p])

    def drain_stores(p):
        for b in range(BATCH):
            pltpu.make_async_copy(
                bufsets[p][b], out_hbm.at[pl.ds(0, CHUNK)], ssems[p]).wait()

    start_set(0, 0)

    @pl.loop(0, NQ, step=2)
    def quad_pair(q):
        for t in range(2):
            qq = q + t

            @pl.when(qq >= 1)
            def _(t=t):
                drain_stores(1 - t)

            @pl.when(qq + 1 < NQ)
            def _(qq=qq, t=t):
                start_set(qq + 1, 1 - t)

            drain_gathers(t)

            def add_row(r, carry, t=t):
                pbuf = posbufs[t]
                bufs = bufsets[t]
                for c in range(SLICES):
                    sl = pl.ds(c * LANES, LANES)
                    pv = pbuf[r, sl]
                    for b in range(BATCH):
                        plsc.addupdate(bufs[b].at[r, sl], pv)
                return carry

            lax.fori_loop(0, CHUNK, add_row, 0, unroll=False)
            start_stores(qq, t)

    drain_stores((NQ - 1) % 2)


@jax.jit
def _embed(flat_ids, word_embeddings, position_embeddings):
    mesh = plsc.VectorSubcoreMesh(core_axis_name="c", subcore_axis_name="s")
    buf = pltpu.VMEM((CHUNK, DIM), jnp.float32)
    return pl.kernel(
        _body,
        out_type=jax.ShapeDtypeStruct((N, DIM), jnp.float32),
        mesh=mesh,
        scratch_types=[
            pltpu.VMEM((BATCH * SPW,), jnp.int32),
            buf, buf,
            buf, buf, buf, buf, buf, buf, buf, buf,
            pltpu.SemaphoreType.DMA, pltpu.SemaphoreType.DMA,
            pltpu.SemaphoreType.DMA, pltpu.SemaphoreType.DMA,
        ],
    )(flat_ids, word_embeddings, position_embeddings)


def kernel(input_ids, word_embeddings, position_embeddings):
    flat_ids = input_ids.reshape(-1).astype(jnp.int32)
    out = _embed(flat_ids, word_embeddings, position_embeddings)
    return out.reshape(BATCH, SEQLEN, DIM)
